# Initial kernel scaffold; baseline (speedup 1.0000x reference)
import jax
import jax.numpy as jnp
from jax.experimental import pallas as pl


def kernel(sender_idx, receiver_idx, commentary_weight):
    raise NotImplementedError("write your pallas kernel here")


# speedup vs baseline: 1.1662x; 1.1662x over previous
import functools

import jax
import jax.numpy as jnp
from jax import lax
from jax.experimental import pallas as pl
from jax.experimental.pallas import tpu as pltpu
from jax.experimental.pallas import tpu_sc as plsc

_S = 1000
_R = 1000
_B = 16384
_NC = 2
_NS = 16
_NW = _NC * _NS
_BPW = _B // _NW
_L = 16


def _stats_body(w_ref, m_ref, z_ref):
    x = w_ref[...]
    m = jnp.max(x)
    z = jnp.sum(jnp.exp(x - m))
    for j in range(_L):
        m_ref[j] = m
        z_ref[j] = z


_stats_tc = pl.pallas_call(
    _stats_body,
    out_shape=(
        jax.ShapeDtypeStruct((_L,), jnp.float32),
        jax.ShapeDtypeStruct((_L,), jnp.float32),
    ),
    in_specs=[pl.BlockSpec(memory_space=pltpu.VMEM)],
    out_specs=(
        pl.BlockSpec(memory_space=pltpu.SMEM),
        pl.BlockSpec(memory_space=pltpu.SMEM),
    ),
)


_sc_mesh = plsc.VectorSubcoreMesh(core_axis_name="c", subcore_axis_name="s")


@functools.partial(
    pl.kernel,
    out_type=jax.ShapeDtypeStruct((_B,), jnp.float32),
    mesh=_sc_mesh,
    scratch_types=[
        pltpu.VMEM((_BPW,), jnp.int32),
        pltpu.VMEM((_BPW,), jnp.int32),
        pltpu.VMEM((4, 128), jnp.int32),
        pltpu.VMEM((4, 128), jnp.float32),
        pltpu.VMEM((_L,), jnp.float32),
        pltpu.VMEM((_L,), jnp.float32),
        pltpu.VMEM((_BPW,), jnp.float32),
        pltpu.SemaphoreType.DMA,
    ],
)
def _gather_sc(snd_hbm, rcv_hbm, tbl_hbm, m_hbm, z_hbm, out_hbm,
               snd_v, rcv_v, idx_v, g_v, m_v, z_v, out_v, sem):
    wid = lax.axis_index("s") * _NC + lax.axis_index("c")
    base = wid * _BPW

    pltpu.sync_copy(snd_hbm.at[pl.ds(base, _BPW)], snd_v)
    pltpu.sync_copy(rcv_hbm.at[pl.ds(base, _BPW)], rcv_v)
    pltpu.sync_copy(m_hbm, m_v)
    pltpu.sync_copy(z_hbm, z_v)

    for j in range(4):
        for k in range(8):
            off = j * 128 + k * 16
            sv = snd_v[pl.ds(off, _L)]
            rv = rcv_v[pl.ds(off, _L)]
            idx_v[j, pl.ds(k * 16, _L)] = sv * _R + rv

    copies = [
        pltpu.async_copy(tbl_hbm.at[idx_v.at[j]], g_v.at[j], sem)
        for j in range(4)
    ]
    for c in copies:
        c.wait()

    mv = m_v[...]
    zv = z_v[...]
    for j in range(4):
        for k in range(8):
            g = g_v[j, pl.ds(k * 16, _L)]
            out_v[pl.ds(j * 128 + k * 16, _L)] = jnp.exp(g - mv) / zv

    pltpu.sync_copy(out_v, out_hbm.at[pl.ds(base, _BPW)])


def kernel(sender_idx, receiver_idx, commentary_weight):
    w2d = commentary_weight.reshape(_S, _R)
    m16, z16 = _stats_tc(w2d)
    table = commentary_weight.reshape(_S * _R)
    return _gather_sc(sender_idx.astype(jnp.int32),
                      receiver_idx.astype(jnp.int32),
                      table, m16, z16)


# speedup vs baseline: 2.9577x; 2.5361x over previous
import functools

import jax
import jax.numpy as jnp
from jax import lax
from jax.experimental import pallas as pl
from jax.experimental.pallas import tpu as pltpu
from jax.experimental.pallas import tpu_sc as plsc

_S = 1000
_R = 1000
_B = 16384
_N = _S * _R
_NC = 1
_NS = 16
_NW = _NC * _NS
_BPW = _B // _NW
_JB = _BPW // 128
_L = 16

_sc_mesh = plsc.VectorSubcoreMesh(core_axis_name="c", subcore_axis_name="s",
                                  num_cores=_NC)


@functools.partial(
    pl.kernel,
    out_type=jax.ShapeDtypeStruct((_B,), jnp.float32),
    mesh=_sc_mesh,
    scratch_types=[
        pltpu.VMEM((_BPW,), jnp.int32),
        pltpu.VMEM((_BPW,), jnp.int32),
        pltpu.VMEM((_JB, 128), jnp.int32),
        pltpu.VMEM((_JB, 128), jnp.float32),
        pltpu.SemaphoreType.DMA,
        pltpu.SemaphoreType.DMA,
    ],
)
def _sc_gather(snd_hbm, rcv_hbm, wt_hbm, out_hbm,
               snd_v, rcv_v, idx_v, g_v, sem_io, sem_g):
    c = lax.axis_index("c")
    t = lax.axis_index("s")
    wid = t * _NC + c
    base = wid * _BPW
    tbl = wt_hbm.at[0]

    cp_s = pltpu.async_copy(snd_hbm.at[pl.ds(base, _BPW)], snd_v, sem_io)
    cp_r = pltpu.async_copy(rcv_hbm.at[pl.ds(base, _BPW)], rcv_v, sem_io)
    cp_s.wait()
    cp_r.wait()

    for j in range(_JB):
        for k in range(8):
            o = j * 128 + k * _L
            idx_v[j, pl.ds(k * _L, _L)] = snd_v[pl.ds(o, _L)] * _R + rcv_v[pl.ds(o, _L)]

    gathers = [
        pltpu.async_copy(tbl.at[idx_v.at[j]], g_v.at[j], sem_g)
        for j in range(_JB)
    ]
    for j, cp in enumerate(gathers):
        cp.wait()
        pltpu.sync_copy(g_v.at[j], out_hbm.at[pl.ds(base + j * 128, 128)])


def _stats_body(w_ref, m_ref, z_ref):
    x = w_ref[...]
    m = jnp.max(x)
    z = jnp.sum(jnp.exp(x - m))
    for j in range(_L):
        m_ref[j] = m
        z_ref[j] = z


_stats_tc = pl.pallas_call(
    _stats_body,
    out_shape=(
        jax.ShapeDtypeStruct((_L,), jnp.float32),
        jax.ShapeDtypeStruct((_L,), jnp.float32),
    ),
    in_specs=[pl.BlockSpec(memory_space=pltpu.VMEM)],
    out_specs=(
        pl.BlockSpec(memory_space=pltpu.SMEM),
        pl.BlockSpec(memory_space=pltpu.SMEM),
    ),
)


def _finalize_body(g_ref, m_ref, z_ref, o_ref):
    o_ref[...] = jnp.exp(g_ref[...] - m_ref[0]) * (1.0 / z_ref[0])


_finalize_tc = pl.pallas_call(
    _finalize_body,
    out_shape=jax.ShapeDtypeStruct((_B,), jnp.float32),
    in_specs=[
        pl.BlockSpec(memory_space=pltpu.VMEM),
        pl.BlockSpec(memory_space=pltpu.SMEM),
        pl.BlockSpec(memory_space=pltpu.SMEM),
    ],
    out_specs=pl.BlockSpec(memory_space=pltpu.VMEM),
)


def kernel(sender_idx, receiver_idx, commentary_weight):
    wt = commentary_weight.T
    g_raw = _sc_gather(sender_idx.astype(jnp.int32),
                       receiver_idx.astype(jnp.int32), wt)
    m16, z16 = _stats_tc(wt)
    return _finalize_tc(g_raw, m16, z16)
